# Initial kernel scaffold; baseline (speedup 1.0000x reference)
import jax
import jax.numpy as jnp
from jax.experimental import pallas as pl


def kernel(x, W, b):
    raise NotImplementedError("write your pallas kernel here")


# speedup vs baseline: 7.5364x; 7.5364x over previous
import functools

import jax
import jax.numpy as jnp
from jax.experimental import pallas as pl

_TOPK = 8
_NE = 64
_BLK = 512
_NEG = float("-inf")


def _router_kernel(x_ref, w_ref, b_ref, router_ref, idx_ref):
    x = x_ref[...]
    w = w_ref[...]
    logits_t = jax.lax.dot_general(
        w, x, (((1,), (1,)), ((), ())),
        preferred_element_type=jnp.float32)
    logits_t = logits_t + b_ref[...]

    iota_e = jax.lax.broadcasted_iota(jnp.int32, logits_t.shape, 0)
    work = logits_t
    mask = jnp.zeros(logits_t.shape, jnp.bool_)
    idx_rows = []
    maxv = None
    for k in range(_TOPK):
        m = jnp.max(work, axis=0, keepdims=True)
        if k == 0:
            maxv = m
        is_m = work == m
        idx = jnp.min(jnp.where(is_m, iota_e, _NE), axis=0,
                      keepdims=True)
        sel = iota_e == idx
        idx_rows.append(idx)
        mask = jnp.logical_or(mask, sel)
        work = jnp.where(sel, _NEG, work)

    ex = jnp.where(mask, jnp.exp(logits_t - maxv), jnp.float32(0.0))
    denom = jnp.sum(ex, axis=0, keepdims=True)
    router_t = ex / denom
    router_ref[...] = router_t.T
    idx_t = jnp.concatenate(idx_rows, axis=0)
    idx_ref[...] = idx_t.T


@jax.jit
def kernel(x, W, b):
    n_rows = x.shape[0]
    grid = (n_rows // _BLK,)
    router, idx = pl.pallas_call(
        _router_kernel,
        grid=grid,
        in_specs=[
            pl.BlockSpec((_BLK, x.shape[1]), lambda i: (i, 0)),
            pl.BlockSpec((_NE, x.shape[1]), lambda i: (0, 0)),
            pl.BlockSpec((_NE, 1), lambda i: (0, 0)),
        ],
        out_specs=[
            pl.BlockSpec((_BLK, _NE), lambda i: (i, 0)),
            pl.BlockSpec((_BLK, _TOPK), lambda i: (i, 0)),
        ],
        out_shape=[
            jax.ShapeDtypeStruct((n_rows, _NE), jnp.float32),
            jax.ShapeDtypeStruct((n_rows, _TOPK), jnp.int32),
        ],
    )(x, W, b.reshape(_NE, 1))
    return router, idx
